# baseline (speedup 1.0000x reference)
import jax
import jax.numpy as jnp
from jax import lax
from jax.experimental import pallas as pl
from jax.experimental.pallas import tpu as pltpu
from jax.experimental.pallas import tpu_sc as plsc

_BATCH = 16384
_FEAT = 64
_NC = 2
_NS = 16
_NW = _NC * _NS
_BPW = _BATCH // _NW
_IDX_CHUNK = 128
_NCHUNK = _BPW // _IDX_CHUNK


def _body(emb_hbm, idx_hbm, wsel_hbm, cen_hbm, out_hbm,
          idx_v, cen_v, emb_v, w_v, acc_v, gsem, esem, wsem):
    wid = lax.axis_index("s") * _NC + lax.axis_index("c")
    base = wid * _BPW

    pltpu.sync_copy(idx_hbm.at[pl.ds(wid * _NCHUNK, _NCHUNK)], idx_v)
    ecopy = pltpu.async_copy(
        emb_hbm.at[pl.ds(base * _FEAT, _BPW * _FEAT)], emb_v, esem)
    wcopy = pltpu.async_copy(
        wsel_hbm.at[pl.ds(base * 16, _BPW * 16)], w_v, wsem)
    gathers = [
        pltpu.async_copy(
            cen_hbm.at[idx_v.at[j]],
            cen_v.at[pl.ds(j * _IDX_CHUNK, _IDX_CHUNK)],
            gsem,
        )
        for j in range(_NCHUNK)
    ]
    ecopy.wait()
    wcopy.wait()

    acc = jnp.zeros((16,), jnp.float32)
    for j in range(_NCHUNK):
        gathers[j].wait()

        def step(i, acc, j=j):
            r = j * _IDX_CHUNK + i
            w = w_v[pl.ds(r * 16, 16)]
            for c in range(_FEAT // 16):
                e = emb_v[pl.ds(r * _FEAT + c * 16, 16)]
                gl = cen_v[r, pl.ds(c * 16, 16)]
                gr = cen_v[r, pl.ds(_FEAT + c * 16, 16)]
                g = gl + w * (gr - gl)
                d = e - g
                acc = acc + d * d
            return acc

        acc = lax.fori_loop(0, _IDX_CHUNK, step, acc)

    acc_v[...] = acc * (1.0 / (2.0 * _BATCH))
    pltpu.sync_copy(acc_v, out_hbm.at[wid])


@jax.jit
def _center_loss(embeddings, labels, centers):
    lab = labels.astype(jnp.int32)
    idx2 = (lab >> 1).reshape(_BATCH // _IDX_CHUNK, _IDX_CHUNK)
    wsel = jnp.repeat((lab & 1).astype(jnp.float32), 16)
    emb1 = embeddings.reshape(-1)
    cen2 = centers.reshape(centers.shape[0] // 2, 2 * _FEAT)
    kern = pl.kernel(
        _body,
        out_type=jax.ShapeDtypeStruct((_NW, 16), jnp.float32),
        mesh=plsc.VectorSubcoreMesh(core_axis_name="c", subcore_axis_name="s"),
        scratch_types=[
            pltpu.VMEM((_NCHUNK, _IDX_CHUNK), jnp.int32),
            pltpu.VMEM((_BPW, 128), jnp.float32),
            pltpu.VMEM((_BPW * _FEAT,), jnp.float32),
            pltpu.VMEM((_BPW * 16,), jnp.float32),
            pltpu.VMEM((16,), jnp.float32),
            pltpu.SemaphoreType.DMA,
            pltpu.SemaphoreType.DMA,
            pltpu.SemaphoreType.DMA,
        ],
    )
    partials = kern(emb1, idx2, wsel, cen2)
    return jnp.sum(partials)


def kernel(embeddings, labels, centers):
    return _center_loss(embeddings, labels, centers)


# speedup vs baseline: 2.1256x; 2.1256x over previous
import jax
import jax.numpy as jnp
from jax import lax
from jax.experimental import pallas as pl
from jax.experimental.pallas import tpu as pltpu
from jax.experimental.pallas import tpu_sc as plsc

_BATCH = 16384
_FEAT = 64
_CLASSES = 100000
_NW = 32
_FPW = _FEAT // _NW
_HALF = _BATCH // 2


def _body(embT_hbm, lab_hbm, cenT_hbm, out_hbm,
          row_v, lab_v, emb_v, acc_v, sem):
    w = lax.axis_index("s") * 2 + lax.axis_index("c")

    pltpu.sync_copy(lab_hbm, lab_v)

    acc = jnp.zeros((16,), jnp.float32)
    for k in range(_FPW):
        f = w * _FPW + k
        pltpu.sync_copy(cenT_hbm.at[f], row_v)
        for h in range(2):
            pltpu.sync_copy(embT_hbm.at[f, pl.ds(h * _HALF, _HALF)], emb_v)

            def step(t, acc, h=h):
                idx16 = lab_v[pl.ds(h * _HALF + t * 16, 16)]
                g = plsc.load_gather(row_v, [idx16])
                e = emb_v[pl.ds(t * 16, 16)]
                d = e - g
                return acc + d * d

            acc = lax.fori_loop(0, _HALF // 16, step, acc)

    acc_v[...] = acc * (1.0 / (2.0 * _BATCH))
    pltpu.sync_copy(acc_v, out_hbm.at[w])


@jax.jit
def _center_loss(embeddings, labels, centers):
    lab = labels.astype(jnp.int32)
    embT = embeddings.T
    cenT = centers.T
    kern = pl.kernel(
        _body,
        out_type=jax.ShapeDtypeStruct((_NW, 16), jnp.float32),
        mesh=plsc.VectorSubcoreMesh(core_axis_name="c", subcore_axis_name="s"),
        scratch_types=[
            pltpu.VMEM((_CLASSES,), jnp.float32),
            pltpu.VMEM((_BATCH,), jnp.int32),
            pltpu.VMEM((_HALF,), jnp.float32),
            pltpu.VMEM((16,), jnp.float32),
            pltpu.SemaphoreType.DMA,
        ],
        compiler_params=pltpu.CompilerParams(needs_layout_passes=False),
    )
    partials = kern(embT, lab, cenT)
    return jnp.sum(partials)


def kernel(embeddings, labels, centers):
    return _center_loss(embeddings, labels, centers)


# speedup vs baseline: 2.4891x; 1.1710x over previous
import jax
import jax.numpy as jnp
from jax import lax
from jax.experimental import pallas as pl
from jax.experimental.pallas import tpu as pltpu
from jax.experimental.pallas import tpu_sc as plsc

_BATCH = 16384
_FEAT = 64
_CLASSES = 100000
_NW = 32
_FPW = _FEAT // _NW
_HALF = _BATCH // 2


def _body(embT_hbm, lab_hbm, cenT_hbm, out_hbm,
          row_v, lab_v, emb_v, acc_v, sem):
    w = lax.axis_index("s") * 2 + lax.axis_index("c")

    pltpu.sync_copy(lab_hbm, lab_v)

    zero = jnp.zeros((16,), jnp.float32)
    accs = (zero, zero, zero, zero)
    for k in range(_FPW):
        f = w * _FPW + k
        pltpu.sync_copy(cenT_hbm.at[f], row_v)
        for h in range(2):
            pltpu.sync_copy(embT_hbm.at[f, pl.ds(h * _HALF, _HALF)], emb_v)

            def step(t, accs, h=h):
                out = []
                for u in range(4):
                    idx16 = lab_v[pl.ds(h * _HALF + t * 64 + u * 16, 16)]
                    g = plsc.load_gather(row_v, [idx16])
                    e = emb_v[pl.ds(t * 64 + u * 16, 16)]
                    d = e - g
                    out.append(accs[u] + d * d)
                return tuple(out)

            accs = lax.fori_loop(0, _HALF // 64, step, accs)

    acc = (accs[0] + accs[1]) + (accs[2] + accs[3])
    acc_v[...] = acc * (1.0 / (2.0 * _BATCH))
    pltpu.sync_copy(acc_v, out_hbm.at[w])


@jax.jit
def _center_loss(embeddings, labels, centers):
    lab = labels.astype(jnp.int32)
    embT = embeddings.T
    cenT = centers.T
    kern = pl.kernel(
        _body,
        out_type=jax.ShapeDtypeStruct((_NW, 16), jnp.float32),
        mesh=plsc.VectorSubcoreMesh(core_axis_name="c", subcore_axis_name="s"),
        scratch_types=[
            pltpu.VMEM((_CLASSES,), jnp.float32),
            pltpu.VMEM((_BATCH,), jnp.int32),
            pltpu.VMEM((_HALF,), jnp.float32),
            pltpu.VMEM((16,), jnp.float32),
            pltpu.SemaphoreType.DMA,
        ],
        compiler_params=pltpu.CompilerParams(needs_layout_passes=False),
    )
    partials = kern(embT, lab, cenT)
    return jnp.sum(partials)


def kernel(embeddings, labels, centers):
    return _center_loss(embeddings, labels, centers)


# speedup vs baseline: 2.4946x; 1.0022x over previous
import jax
import jax.numpy as jnp
from jax import lax
from jax.experimental import pallas as pl
from jax.experimental.pallas import tpu as pltpu
from jax.experimental.pallas import tpu_sc as plsc

_BATCH = 16384
_FEAT = 64
_CLASSES = 100000
_NW = 32
_FPW = _FEAT // _NW
_HALF = _BATCH // 2


def _body(embT_hbm, lab_hbm, cenT_hbm, out_hbm,
          row_v, lab_v, emb_v, acc_v, sem):
    w = lax.axis_index("s") * 2 + lax.axis_index("c")

    pltpu.sync_copy(lab_hbm, lab_v)

    zero = jnp.zeros((16,), jnp.float32)
    accs = (zero, zero, zero, zero)
    for k in range(_FPW):
        f = w * _FPW + k
        pltpu.sync_copy(cenT_hbm.at[f], row_v)
        for h in range(2):
            pltpu.sync_copy(embT_hbm.at[f, pl.ds(h * _HALF, _HALF)], emb_v)

            @plsc.parallel_loop(0, _HALF, step=64, carry=accs)
            def accs(t, accs, h=h):
                out = []
                for u in range(4):
                    idx16 = lab_v[pl.ds(h * _HALF + t + u * 16, 16)]
                    g = plsc.load_gather(row_v, [idx16])
                    e = emb_v[pl.ds(t + u * 16, 16)]
                    d = e - g
                    out.append(accs[u] + d * d)
                return tuple(out)

    acc = (accs[0] + accs[1]) + (accs[2] + accs[3])
    acc_v[...] = acc * (1.0 / (2.0 * _BATCH))
    pltpu.sync_copy(acc_v, out_hbm.at[w])


@jax.jit
def _center_loss(embeddings, labels, centers):
    lab = labels.astype(jnp.int32)
    embT = embeddings.T
    cenT = centers.T
    kern = pl.kernel(
        _body,
        out_type=jax.ShapeDtypeStruct((_NW, 16), jnp.float32),
        mesh=plsc.VectorSubcoreMesh(core_axis_name="c", subcore_axis_name="s"),
        scratch_types=[
            pltpu.VMEM((_CLASSES,), jnp.float32),
            pltpu.VMEM((_BATCH,), jnp.int32),
            pltpu.VMEM((_HALF,), jnp.float32),
            pltpu.VMEM((16,), jnp.float32),
            pltpu.SemaphoreType.DMA,
        ],
        compiler_params=pltpu.CompilerParams(needs_layout_passes=False),
    )
    partials = kern(embT, lab, cenT)
    return jnp.sum(partials)


def kernel(embeddings, labels, centers):
    return _center_loss(embeddings, labels, centers)


# speedup vs baseline: 2.6895x; 1.0781x over previous
import jax
import jax.numpy as jnp
from jax import lax
from jax.experimental import pallas as pl
from jax.experimental.pallas import tpu as pltpu
from jax.experimental.pallas import tpu_sc as plsc

_BATCH = 16384
_FEAT = 64
_CLASSES = 100000
_NW = 32
_FPW = _FEAT // _NW
_QTR = _BATCH // 4


def _body(embT_hbm, lab_hbm, cenT_hbm, out_hbm,
          row_v, lab_v, emb_v, acc_v, lab_sh, sem, esem):
    cid = lax.axis_index("c")
    sid = lax.axis_index("s")
    w = sid * 2 + cid

    @pl.when(sid == 0)
    def _():
        pltpu.sync_copy(lab_hbm, lab_sh)

    plsc.subcore_barrier()
    pltpu.sync_copy(lab_sh, lab_v)

    zero = jnp.zeros((16,), jnp.float32)
    accs = (zero, zero, zero, zero)
    for k in range(_FPW):
        f = w * _FPW + k
        pltpu.sync_copy(cenT_hbm.at[f], row_v)
        ecopies = [None, None, None, None]
        ecopies[0] = pltpu.async_copy(
            embT_hbm.at[f, pl.ds(0, _QTR)], emb_v.at[0], esem)
        for q in range(4):
            if q < 3:
                ecopies[q + 1] = pltpu.async_copy(
                    embT_hbm.at[f, pl.ds((q + 1) * _QTR, _QTR)],
                    emb_v.at[(q + 1) % 2], esem)
            ecopies[q].wait()

            @plsc.parallel_loop(0, _QTR, step=64, carry=accs)
            def accs(t, accs, q=q):
                out = []
                for u in range(4):
                    idx16 = lab_v[pl.ds(q * _QTR + t + u * 16, 16)]
                    g = plsc.load_gather(row_v, [idx16])
                    e = emb_v[q % 2, pl.ds(t + u * 16, 16)]
                    d = e - g
                    out.append(accs[u] + d * d)
                return tuple(out)

    acc = (accs[0] + accs[1]) + (accs[2] + accs[3])
    acc_v[...] = acc * (1.0 / (2.0 * _BATCH))
    pltpu.sync_copy(acc_v, out_hbm.at[w])


@jax.jit
def _center_loss(embeddings, labels, centers):
    lab = labels.astype(jnp.int32)
    embT = embeddings.T
    cenT = centers.T
    kern = pl.kernel(
        _body,
        out_type=jax.ShapeDtypeStruct((_NW, 16), jnp.float32),
        mesh=plsc.VectorSubcoreMesh(core_axis_name="c", subcore_axis_name="s"),
        scratch_types=[
            pltpu.VMEM((_CLASSES,), jnp.float32),
            pltpu.VMEM((_BATCH,), jnp.int32),
            pltpu.VMEM((2, _QTR), jnp.float32),
            pltpu.VMEM((16,), jnp.float32),
            pltpu.VMEM_SHARED((_BATCH,), jnp.int32),
            pltpu.SemaphoreType.DMA,
            pltpu.SemaphoreType.DMA,
        ],
        compiler_params=pltpu.CompilerParams(needs_layout_passes=False),
    )
    partials = kern(embT, lab, cenT)
    return jnp.sum(partials)


def kernel(embeddings, labels, centers):
    return _center_loss(embeddings, labels, centers)


# speedup vs baseline: 2.7372x; 1.0177x over previous
import jax
import jax.numpy as jnp
from jax import lax
from jax.experimental import pallas as pl
from jax.experimental.pallas import tpu as pltpu
from jax.experimental.pallas import tpu_sc as plsc

_BATCH = 16384
_FEAT = 64
_CLASSES = 100000
_NW = 32
_FPW = _FEAT // _NW
_QTR = _BATCH // 4


def _body(embT_hbm, lab_hbm, cenT_hbm, out_hbm,
          row_v, lab_v, emb_v, acc_v, lab_sh, sem, esem):
    cid = lax.axis_index("c")
    sid = lax.axis_index("s")
    w = sid * 2 + cid

    @pl.when(sid == 0)
    def _():
        pltpu.sync_copy(lab_hbm, lab_sh)

    plsc.subcore_barrier()
    pltpu.sync_copy(lab_sh, lab_v)

    zero = jnp.zeros((16,), jnp.float32)
    accs = (zero, zero, zero, zero)

    def feat_body(k, accs):
        f = w * _FPW + k
        pltpu.sync_copy(cenT_hbm.at[f], row_v)
        pltpu.async_copy(embT_hbm.at[f, pl.ds(0, _QTR)], emb_v.at[0], esem)

        def qtr_body(q, accs):
            qmod = lax.rem(q, 2)

            @pl.when(q < 3)
            def _():
                pltpu.async_copy(
                    embT_hbm.at[f, pl.ds((q + 1) * _QTR, _QTR)],
                    emb_v.at[1 - qmod], esem)

            pltpu.make_async_copy(
                embT_hbm.at[f, pl.ds(0, _QTR)], emb_v.at[0], esem).wait()

            @plsc.parallel_loop(0, _QTR, step=64, carry=accs)
            def accs(t, accs):
                out = []
                for u in range(4):
                    idx16 = lab_v[pl.ds(q * _QTR + t + u * 16, 16)]
                    g = plsc.load_gather(row_v, [idx16])
                    e = emb_v[qmod, pl.ds(t + u * 16, 16)]
                    d = e - g
                    out.append(accs[u] + d * d)
                return tuple(out)

            return accs

        return lax.fori_loop(0, 4, qtr_body, accs)

    accs = lax.fori_loop(0, _FPW, feat_body, accs)

    acc = (accs[0] + accs[1]) + (accs[2] + accs[3])
    acc_v[...] = acc * (1.0 / (2.0 * _BATCH))
    pltpu.sync_copy(acc_v, out_hbm.at[w])


@jax.jit
def _center_loss(embeddings, labels, centers):
    lab = labels.astype(jnp.int32)
    embT = embeddings.T
    cenT = centers.T
    kern = pl.kernel(
        _body,
        out_type=jax.ShapeDtypeStruct((_NW, 16), jnp.float32),
        mesh=plsc.VectorSubcoreMesh(core_axis_name="c", subcore_axis_name="s"),
        scratch_types=[
            pltpu.VMEM((_CLASSES,), jnp.float32),
            pltpu.VMEM((_BATCH,), jnp.int32),
            pltpu.VMEM((2, _QTR), jnp.float32),
            pltpu.VMEM((16,), jnp.float32),
            pltpu.VMEM_SHARED((_BATCH,), jnp.int32),
            pltpu.SemaphoreType.DMA,
            pltpu.SemaphoreType.DMA,
        ],
        compiler_params=pltpu.CompilerParams(needs_layout_passes=False),
    )
    partials = kern(embT, lab, cenT)
    return jnp.sum(partials)


def kernel(embeddings, labels, centers):
    return _center_loss(embeddings, labels, centers)


# speedup vs baseline: 2.9539x; 1.0792x over previous
import jax
import jax.numpy as jnp
from jax import lax
from jax.experimental import pallas as pl
from jax.experimental.pallas import tpu as pltpu
from jax.experimental.pallas import tpu_sc as plsc

_BATCH = 16384
_FEAT = 64
_CLASSES = 100000
_NW = 32
_FPW = _FEAT // _NW
_QTR = _BATCH // 4


def _body(embT_hbm, lab_hbm, cenT_hbm, out_hbm,
          row_v, lab_v, emb_v, acc_v, lab_sh, sem, esem):
    cid = lax.axis_index("c")
    sid = lax.axis_index("s")
    w = sid * 2 + cid

    pltpu.async_copy(cenT_hbm.at[w * _FPW], row_v, sem)
    pltpu.async_copy(embT_hbm.at[w * _FPW, pl.ds(0, _QTR)], emb_v.at[0], esem)

    @pl.when(sid == 0)
    def _():
        pltpu.sync_copy(lab_hbm, lab_sh)

    plsc.subcore_barrier()
    pltpu.sync_copy(lab_sh, lab_v)

    zero = jnp.zeros((16,), jnp.float32)
    accs = (zero, zero, zero, zero)

    def feat_body(k, accs):
        f = w * _FPW + k

        @pl.when(k > 0)
        def _():
            pltpu.async_copy(embT_hbm.at[f, pl.ds(0, _QTR)], emb_v.at[0], esem)
            pltpu.async_copy(cenT_hbm.at[f], row_v, sem)

        pltpu.make_async_copy(cenT_hbm.at[f], row_v, sem).wait()

        def qtr_body(q, accs):
            qmod = lax.rem(q, 2)

            @pl.when(q < 3)
            def _():
                pltpu.async_copy(
                    embT_hbm.at[f, pl.ds((q + 1) * _QTR, _QTR)],
                    emb_v.at[1 - qmod], esem)

            pltpu.make_async_copy(
                embT_hbm.at[f, pl.ds(0, _QTR)], emb_v.at[0], esem).wait()

            @plsc.parallel_loop(0, _QTR, step=64, carry=accs)
            def accs(t, accs):
                out = []
                for u in range(4):
                    idx16 = lab_v[pl.ds(q * _QTR + t + u * 16, 16)]
                    g = plsc.load_gather(row_v, [idx16])
                    e = emb_v[qmod, pl.ds(t + u * 16, 16)]
                    d = e - g
                    out.append(accs[u] + d * d)
                return tuple(out)

            return accs

        return lax.fori_loop(0, 4, qtr_body, accs)

    accs = lax.fori_loop(0, _FPW, feat_body, accs)

    acc = (accs[0] + accs[1]) + (accs[2] + accs[3])
    acc_v[...] = acc * (1.0 / (2.0 * _BATCH))
    pltpu.sync_copy(acc_v, out_hbm.at[w])


@jax.jit
def _center_loss(embeddings, labels, centers):
    lab = labels.astype(jnp.int32)
    embT = embeddings.T
    cenT = centers.T
    kern = pl.kernel(
        _body,
        out_type=jax.ShapeDtypeStruct((_NW, 16), jnp.float32),
        mesh=plsc.VectorSubcoreMesh(core_axis_name="c", subcore_axis_name="s"),
        scratch_types=[
            pltpu.VMEM((_CLASSES,), jnp.float32),
            pltpu.VMEM((_BATCH,), jnp.int32),
            pltpu.VMEM((2, _QTR), jnp.float32),
            pltpu.VMEM((16,), jnp.float32),
            pltpu.VMEM_SHARED((_BATCH,), jnp.int32),
            pltpu.SemaphoreType.DMA,
            pltpu.SemaphoreType.DMA,
        ],
        compiler_params=pltpu.CompilerParams(needs_layout_passes=False),
    )
    partials = kern(embT, lab, cenT)
    return jnp.sum(partials)


def kernel(embeddings, labels, centers):
    return _center_loss(embeddings, labels, centers)


# speedup vs baseline: 2.9699x; 1.0054x over previous
import jax
import jax.numpy as jnp
from jax import lax
from jax.experimental import pallas as pl
from jax.experimental.pallas import tpu as pltpu
from jax.experimental.pallas import tpu_sc as plsc

_BATCH = 16384
_FEAT = 64
_CLASSES = 100000
_NW = 32
_FPW = _FEAT // _NW
_QTR = _BATCH // 4


def _body(embT_hbm, lab_hbm, cenT_hbm, out_hbm,
          row_v, lab_v, emb_v, acc_v, lab_sh, sem, esem):
    cid = lax.axis_index("c")
    sid = lax.axis_index("s")
    w = sid * 2 + cid

    pltpu.async_copy(cenT_hbm.at[w * _FPW], row_v, sem)
    pltpu.async_copy(embT_hbm.at[w * _FPW, pl.ds(0, _QTR)], emb_v.at[0], esem)

    @pl.when(sid == 0)
    def _():
        pltpu.sync_copy(lab_hbm, lab_sh)

    plsc.subcore_barrier()
    pltpu.sync_copy(lab_sh, lab_v)

    zero = jnp.zeros((16,), jnp.float32)
    accs = (zero,) * 8

    def feat_body(k, accs):
        f = w * _FPW + k

        @pl.when(k > 0)
        def _():
            pltpu.async_copy(embT_hbm.at[f, pl.ds(0, _QTR)], emb_v.at[0], esem)
            pltpu.async_copy(cenT_hbm.at[f], row_v, sem)

        pltpu.make_async_copy(cenT_hbm.at[f], row_v, sem).wait()

        def qtr_body(q, accs):
            qmod = lax.rem(q, 2)

            @pl.when(q < 3)
            def _():
                pltpu.async_copy(
                    embT_hbm.at[f, pl.ds((q + 1) * _QTR, _QTR)],
                    emb_v.at[1 - qmod], esem)

            pltpu.make_async_copy(
                embT_hbm.at[f, pl.ds(0, _QTR)], emb_v.at[0], esem).wait()

            @plsc.parallel_loop(0, _QTR, step=128, carry=accs)
            def accs(t, accs):
                out = []
                for u in range(8):
                    idx16 = lab_v[pl.ds(q * _QTR + t + u * 16, 16)]
                    g = plsc.load_gather(row_v, [idx16])
                    e = emb_v[qmod, pl.ds(t + u * 16, 16)]
                    d = e - g
                    out.append(accs[u] + d * d)
                return tuple(out)

            return accs

        return lax.fori_loop(0, 4, qtr_body, accs)

    accs = lax.fori_loop(0, _FPW, feat_body, accs)

    acc = ((accs[0] + accs[1]) + (accs[2] + accs[3])
           + (accs[4] + accs[5]) + (accs[6] + accs[7]))
    acc_v[...] = acc * (1.0 / (2.0 * _BATCH))
    pltpu.sync_copy(acc_v, out_hbm.at[w])


@jax.jit
def _center_loss(embeddings, labels, centers):
    lab = labels.astype(jnp.int32)
    embT = embeddings.T
    cenT = centers.T
    kern = pl.kernel(
        _body,
        out_type=jax.ShapeDtypeStruct((_NW, 16), jnp.float32),
        mesh=plsc.VectorSubcoreMesh(core_axis_name="c", subcore_axis_name="s"),
        scratch_types=[
            pltpu.VMEM((_CLASSES,), jnp.float32),
            pltpu.VMEM((_BATCH,), jnp.int32),
            pltpu.VMEM((2, _QTR), jnp.float32),
            pltpu.VMEM((16,), jnp.float32),
            pltpu.VMEM_SHARED((_BATCH,), jnp.int32),
            pltpu.SemaphoreType.DMA,
            pltpu.SemaphoreType.DMA,
        ],
        compiler_params=pltpu.CompilerParams(needs_layout_passes=False),
    )
    partials = kern(embT, lab, cenT)
    return jnp.sum(partials)


def kernel(embeddings, labels, centers):
    return _center_loss(embeddings, labels, centers)
